# baseline (speedup 1.0000x reference)
import functools

import jax
import jax.numpy as jnp
from jax import lax
from jax.experimental import pallas as pl
from jax.experimental.pallas import tpu as pltpu
from jax.experimental.pallas import tpu_sc as plsc


@functools.lru_cache(maxsize=None)
def _make_gather(batch: int, hist: int, vocab: int, dim: int):
  info = plsc.get_sparse_core_info()
  nc, ns, lanes = info.num_cores, info.num_subcores, info.num_lanes
  nw = nc * ns
  n_bt = batch // 128
  assert n_bt % nw == 0 and dim % 8 == 0
  bt_per_w = n_bt // nw
  hg = 10
  assert hist % hg == 0
  mesh = plsc.VectorSubcoreMesh(core_axis_name="c", subcore_axis_name="s")

  @functools.partial(
      pl.kernel,
      mesh=mesh,
      compiler_params=pltpu.CompilerParams(use_tc_tiling_on_sc=False,
                                           needs_layout_passes=False),
      out_type=jax.ShapeDtypeStruct((hist, dim // 8, n_bt, 8, 128),
                                    jnp.float32),
      scratch_types=[
          pltpu.VMEM((hist, 128), jnp.int32),
          pltpu.VMEM((hg * 128, dim), jnp.float32),
          pltpu.VMEM((dim // 8, 8, 128), jnp.float32),
          pltpu.SemaphoreType.DMA,
      ],
  )
  def gather(table_hbm, ids_hbm, out_hbm, idx_v, rows_v, outt_v, sem):
    wid = lax.axis_index("s") * nc + lax.axis_index("c")

    def bt_body(j, carry):
      bt = wid * bt_per_w + j
      pltpu.sync_copy(ids_hbm.at[:, pl.ds(bt * 128, 128)], idx_v)
      for g in range(hist // hg):
        copies = [
            pltpu.async_copy(table_hbm.at[idx_v.at[g * hg + k]],
                             rows_v.at[pl.ds(k * 128, 128), :], sem)
            for k in range(hg)
        ]
        for c in copies:
          c.wait()
        for k in range(hg):

          def d_body(d, carry2, _k=k):
            d_vec = jnp.full((lanes,), d, jnp.int32)
            for s in range(128 // lanes):
              row_vec = lax.iota(jnp.int32, lanes) + (_k * 128 + s * lanes)
              vals = plsc.load_gather(rows_v, [row_vec, d_vec])
              outt_v[d // 8, d % 8, pl.ds(s * lanes, lanes)] = vals
            return carry2

          lax.fori_loop(0, dim, d_body, 0)
          pltpu.sync_copy(outt_v, out_hbm.at[g * hg + k, :, bt, :, :])
      return carry

    lax.fori_loop(0, bt_per_w, bt_body, 0)

  return gather


def kernel(brawler_ids, table):
  batch, hist = brawler_ids.shape
  vocab, dim = table.shape
  out5 = _make_gather(batch, hist, vocab, dim)(
      table, brawler_ids.astype(jnp.int32).T)
  return (out5.transpose(2, 4, 0, 1, 3).reshape(batch, hist, dim))


# speedup vs baseline: 1.0677x; 1.0677x over previous
import functools

import jax
import jax.numpy as jnp
from jax import lax
from jax.experimental import pallas as pl
from jax.experimental.pallas import tpu as pltpu
from jax.experimental.pallas import tpu_sc as plsc


@functools.lru_cache(maxsize=None)
def _make_gather(batch: int, hist: int, vocab: int, dim: int):
  info = plsc.get_sparse_core_info()
  nc, ns, lanes = info.num_cores, info.num_subcores, info.num_lanes
  nw = nc * ns
  n_bt = batch // 128
  assert n_bt % nw == 0 and dim % 8 == 0
  bt_per_w = n_bt // nw
  hg = 10
  n_g = hist // hg
  assert hist % hg == 0
  mesh = plsc.VectorSubcoreMesh(core_axis_name="c", subcore_axis_name="s")

  @functools.partial(
      pl.kernel,
      mesh=mesh,
      compiler_params=pltpu.CompilerParams(use_tc_tiling_on_sc=False,
                                           needs_layout_passes=False),
      out_type=jax.ShapeDtypeStruct((hist, dim // 8, n_bt, 8, 128),
                                    jnp.float32),
      scratch_types=[
          pltpu.VMEM((hist, 128), jnp.int32),
          pltpu.VMEM((2, hg * 128, dim), jnp.float32),
          pltpu.VMEM((2, dim // 8, 8, 128), jnp.float32),
          pltpu.SemaphoreType.DMA,
          pltpu.SemaphoreType.DMA,
      ],
  )
  def gather(table_hbm, ids_hbm, out_hbm, idx_v, rows_v, outt_v, sem_g,
             sem_o):
    wid = lax.axis_index("s") * nc + lax.axis_index("c")

    def fire(g, buf):
      return [
          pltpu.async_copy(table_hbm.at[idx_v.at[g * hg + k]],
                           rows_v.at[buf, pl.ds(k * 128, 128), :], sem_g)
          for k in range(hg)
      ]

    def bt_body(j, carry):
      bt = wid * bt_per_w + j
      pltpu.sync_copy(ids_hbm.at[:, pl.ds(bt * 128, 128)], idx_v)
      in_flight = fire(0, 0)
      out_flight = []
      for g in range(n_g):
        for c in in_flight:
          c.wait()
        if g + 1 < n_g:
          in_flight = fire(g + 1, (g + 1) % 2)

        def k_body(kk, carry2, _g=g):
          buf = _g % 2
          slot = kk % 2

          def d_body(d, carry3, _buf=buf, _kk=kk, _slot=slot):
            for s in range(128 // lanes):
              row_vec = (lax.iota(jnp.int32, lanes)
                         + (_kk * 128 + s * lanes))
              d_vec = jnp.full((lanes,), d, jnp.int32)
              vals = plsc.load_gather(rows_v.at[_buf], [row_vec, d_vec])
              outt_v[_slot, d // 8, d % 8, pl.ds(s * lanes, lanes)] = vals
            return carry3

          lax.fori_loop(0, dim, d_body, 0)
          return carry2

        for kk in range(hg):
          if len(out_flight) >= 2:
            out_flight.pop(0).wait()
          k_body(kk, 0)
          out_flight.append(
              pltpu.async_copy(outt_v.at[kk % 2],
                               out_hbm.at[g * hg + kk, :, bt, :, :],
                               sem_o))
      for c in out_flight:
        c.wait()
      return carry

    lax.fori_loop(0, bt_per_w, bt_body, 0)

  return gather


def kernel(brawler_ids, table):
  batch, hist = brawler_ids.shape
  vocab, dim = table.shape
  out5 = _make_gather(batch, hist, vocab, dim)(
      table, brawler_ids.astype(jnp.int32).T)
  return (out5.transpose(2, 4, 0, 1, 3).reshape(batch, hist, dim))


# speedup vs baseline: 1.6511x; 1.5464x over previous
import functools

import jax
import jax.numpy as jnp
from jax import lax
from jax.experimental import pallas as pl
from jax.experimental.pallas import tpu as pltpu
from jax.experimental.pallas import tpu_sc as plsc


@functools.lru_cache(maxsize=None)
def _make_gather(batch: int, hist: int, vocab: int, dim: int):
  info = plsc.get_sparse_core_info()
  nc, ns, lanes = info.num_cores, info.num_subcores, info.num_lanes
  nw = nc * ns
  n_bt = batch // 128
  assert n_bt % nw == 0 and dim % 8 == 0
  bt_per_w = n_bt // nw
  hg = 10
  n_g = hist // hg
  assert hist % hg == 0
  mesh = plsc.VectorSubcoreMesh(core_axis_name="c", subcore_axis_name="s")

  @functools.partial(
      pl.kernel,
      mesh=mesh,
      compiler_params=pltpu.CompilerParams(use_tc_tiling_on_sc=False,
                                           needs_layout_passes=False),
      out_type=jax.ShapeDtypeStruct((hist, dim // 8, n_bt, 8 * 128),
                                    jnp.float32),
      scratch_types=[
          pltpu.VMEM((hist, 128), jnp.int32),
          pltpu.VMEM((2, hg * 128, dim), jnp.float32),
          pltpu.VMEM((2, dim // 8, 8 * 128), jnp.float32),
          pltpu.SemaphoreType.DMA,
          pltpu.SemaphoreType.DMA,
      ],
  )
  def gather(table_hbm, ids_hbm, out_hbm, idx_v, rows_v, outt_v, sem_g,
             sem_o):
    wid = lax.axis_index("s") * nc + lax.axis_index("c")

    def fire(g, buf):
      return [
          pltpu.async_copy(table_hbm.at[idx_v.at[g * hg + k]],
                           rows_v.at[buf, pl.ds(k * 128, 128), :], sem_g)
          for k in range(hg)
      ]

    iota = lax.iota(jnp.int32, lanes)
    bi_vecs = [iota + s * lanes for s in range(128 // lanes)]

    def bt_body(j, carry):
      bt = wid * bt_per_w + j
      pltpu.sync_copy(ids_hbm.at[:, pl.ds(bt * 128, 128)], idx_v)
      in_flight = fire(0, 0)
      out_flight = []
      for g in range(n_g):
        for c in in_flight:
          c.wait()
        if g + 1 < n_g:
          in_flight = fire(g + 1, (g + 1) % 2)

        def k_body(kk, _g=g):
          buf = _g % 2
          slot = kk % 2
          k_rows = [bv + kk * 128 for bv in bi_vecs]

          def d_body(d, carry3, _buf=buf, _slot=slot, _k_rows=k_rows):
            col = (d + iota) & (dim - 1)
            dt = lax.shift_right_logical(col, 3)
            base_flat = lax.shift_left(col & 7, 7)
            for s in range(128 // lanes):
              vals = plsc.load_gather(rows_v.at[_buf], [_k_rows[s], col])
              plsc.store_scatter(outt_v.at[_slot],
                                 [dt, base_flat + bi_vecs[s]], vals)
            return carry3

          lax.fori_loop(0, dim, d_body, 0)

        for kk in range(hg):
          if len(out_flight) >= 2:
            out_flight.pop(0).wait()
          k_body(kk)
          out_flight.append(
              pltpu.async_copy(outt_v.at[kk % 2],
                               out_hbm.at[g * hg + kk, :, bt, :],
                               sem_o))
      for c in out_flight:
        c.wait()
      return carry

    lax.fori_loop(0, bt_per_w, bt_body, 0)

  return gather


def kernel(brawler_ids, table):
  batch, hist = brawler_ids.shape
  vocab, dim = table.shape
  out5 = _make_gather(batch, hist, vocab, dim)(
      table, brawler_ids.astype(jnp.int32).T)
  out5 = out5.reshape(hist, dim // 8, batch // 128, 8, 128)
  return (out5.transpose(2, 4, 0, 1, 3).reshape(batch, hist, dim))


# speedup vs baseline: 1.6524x; 1.0008x over previous
import functools

import jax
import jax.numpy as jnp
from jax import lax
from jax.experimental import pallas as pl
from jax.experimental.pallas import tpu as pltpu
from jax.experimental.pallas import tpu_sc as plsc


@functools.lru_cache(maxsize=None)
def _make_gather(batch: int, hist: int, vocab: int, dim: int):
  info = plsc.get_sparse_core_info()
  nc, ns, lanes = info.num_cores, info.num_subcores, info.num_lanes
  nw = nc * ns
  n_bt = batch // 128
  assert n_bt % nw == 0 and dim % 8 == 0
  bt_per_w = n_bt // nw
  hg = 10
  n_g = hist // hg
  assert hist % hg == 0
  mesh = plsc.VectorSubcoreMesh(core_axis_name="c", subcore_axis_name="s")

  @functools.partial(
      pl.kernel,
      mesh=mesh,
      compiler_params=pltpu.CompilerParams(use_tc_tiling_on_sc=False,
                                           needs_layout_passes=False),
      out_type=jax.ShapeDtypeStruct((hist, dim // 8, n_bt, 8 * 128),
                                    jnp.float32),
      scratch_types=[
          pltpu.VMEM((hist, 128), jnp.int32),
          pltpu.VMEM((2, hg * 128, dim), jnp.float32),
          pltpu.VMEM((2, dim // 8, 8 * 128), jnp.float32),
          pltpu.SemaphoreType.DMA,
          pltpu.SemaphoreType.DMA,
      ],
  )
  def gather(table_hbm, ids_hbm, out_hbm, idx_v, rows_v, outt_v, sem_g,
             sem_o):
    wid = lax.axis_index("s") * nc + lax.axis_index("c")

    def fire(g, buf):
      return [
          pltpu.async_copy(table_hbm.at[idx_v.at[g * hg + k]],
                           rows_v.at[buf, pl.ds(k * 128, 128), :], sem_g)
          for k in range(hg)
      ]

    iota = lax.iota(jnp.int32, lanes)
    bi_vecs = [iota + s * lanes for s in range(128 // lanes)]

    def bt_body(j, carry):
      bt = wid * bt_per_w + j
      pltpu.sync_copy(ids_hbm.at[:, pl.ds(bt * 128, 128)], idx_v)
      in_flight = fire(0, 0)
      out_flight = []
      for g in range(n_g):
        for c in in_flight:
          c.wait()
        if g + 1 < n_g:
          in_flight = fire(g + 1, (g + 1) % 2)

        def k_body(kk, _g=g):
          buf = _g % 2
          slot = kk % 2
          k_rows = [bv + kk * 128 for bv in bi_vecs]

          def d_body(d, carry3, _buf=buf, _slot=slot, _k_rows=k_rows):
            col = (d + iota) & (dim - 1)
            dt = lax.shift_right_logical(col, 3)
            base_flat = lax.shift_left(col & 7, 7)
            for s in range(128 // lanes):
              vals = plsc.load_gather(rows_v.at[_buf], [_k_rows[s], col])
              plsc.store_scatter(outt_v.at[_slot],
                                 [dt, base_flat + bi_vecs[s]], vals)
            return carry3

          lax.fori_loop(0, dim, d_body, 0)

        for kk in range(hg):
          if len(out_flight) >= 2:
            out_flight.pop(0).wait()
          k_body(kk)
          out_flight.append(
              pltpu.async_copy(outt_v.at[kk % 2],
                               out_hbm.at[g * hg + kk, :, bt, :],
                               sem_o))
      for c in out_flight:
        c.wait()
      return carry

    lax.fori_loop(0, bt_per_w, bt_body, 0)

  return gather


def kernel(brawler_ids, table):
  batch, hist = brawler_ids.shape
  vocab, dim = table.shape
  t128 = lax.optimization_barrier(table.reshape(-1, 128))
  out5 = _make_gather(batch, hist, vocab, dim)(
      t128.reshape(vocab, dim), brawler_ids.astype(jnp.int32).T)
  out5 = out5.reshape(hist, dim // 8, batch // 128, 8, 128)
  return (out5.transpose(2, 4, 0, 1, 3).reshape(batch, hist, dim))
